# baseline (speedup 1.0000x reference)
import math

import jax
import jax.numpy as jnp
from jax.experimental import pallas as pl

_INPUT_SIZE = 1000
_SCALE = 1.0 / math.sqrt(float(_INPUT_SIZE))
_BLOCK_B = 512


def _mapper_block(x_ref, w_ref, o_ref):
    o_ref[...] = (
        jnp.dot(x_ref[...], w_ref[...], preferred_element_type=jnp.float32)
        * _SCALE
        + 1.0
    )


def kernel(x, W):
    B, N = x.shape
    O = W.shape[1]
    grid = (B // _BLOCK_B,)
    return pl.pallas_call(
        _mapper_block,
        grid=grid,
        in_specs=[
            pl.BlockSpec((_BLOCK_B, N), lambda i: (i, 0)),
            pl.BlockSpec((N, O), lambda i: (0, 0)),
        ],
        out_specs=pl.BlockSpec((_BLOCK_B, O), lambda i: (i, 0)),
        out_shape=jax.ShapeDtypeStruct((B, O), jnp.float32),
    )(x, W)


# speedup vs baseline: 1.0071x; 1.0071x over previous
import math

import jax
import jax.numpy as jnp
from jax.experimental import pallas as pl

_INPUT_SIZE = 1000
_SCALE = 1.0 / math.sqrt(float(_INPUT_SIZE))
_BLOCK_B = 256


def _mapper_block(xa_ref, xb_ref, w_ref, o_ref):
    w = w_ref[...]
    o_ref[0:_BLOCK_B, :] = (
        jnp.dot(xa_ref[...], w, preferred_element_type=jnp.float32) * _SCALE
        + 1.0
    )
    o_ref[_BLOCK_B : 2 * _BLOCK_B, :] = (
        jnp.dot(xb_ref[...], w, preferred_element_type=jnp.float32) * _SCALE
        + 1.0
    )


def kernel(x, W):
    B, N = x.shape
    O = W.shape[1]
    grid = (B // (2 * _BLOCK_B),)
    return pl.pallas_call(
        _mapper_block,
        grid=grid,
        in_specs=[
            pl.BlockSpec((_BLOCK_B, N), lambda i: (2 * i, 0)),
            pl.BlockSpec((_BLOCK_B, N), lambda i: (2 * i + 1, 0)),
            pl.BlockSpec((N, O), lambda i: (0, 0)),
        ],
        out_specs=pl.BlockSpec((2 * _BLOCK_B, O), lambda i: (i, 0)),
        out_shape=jax.ShapeDtypeStruct((B, O), jnp.float32),
    )(x, x, W)
